# Initial kernel scaffold; baseline (speedup 1.0000x reference)
import jax
import jax.numpy as jnp
from jax.experimental import pallas as pl


def kernel(x, edge_index, W_r, b_r, W_l, b_l, W_a, b_a):
    raise NotImplementedError("write your pallas kernel here")


# speedup vs baseline: 75.2375x; 75.2375x over previous
import jax
import jax.numpy as jnp
from jax import lax
from jax.experimental import pallas as pl
from jax.experimental.pallas import tpu as pltpu
from jax.experimental.pallas import tpu_sc as plsc

N = 10000
E = 320000
D = 128
HEAD = 4
HD = 32
F = HEAD * HD
FH = F // 2

NC = 2
NS = 16

EPT = E // NS
C = 160
NCHUNK = EPT // C
C2 = 400
NCHUNK2 = EPT // C2
NPAD = 10240
NPT = NPAD // NS
ZR = 128

BN = 1000

_SC_PARAMS = pltpu.CompilerParams(needs_layout_passes=False,
                                  use_tc_tiling_on_sc=False)


def _sc_mesh():
    return plsc.VectorSubcoreMesh(core_axis_name="c", subcore_axis_name="s",
                                  num_cores=NC, num_subcores=NS)


def _prologue_body(x_ref, wl_ref, bl_ref, wr1_ref, wr2_ref, br_ref, wa_ref,
                   sel_ref, h_ref, p_ref, qb_ref, st_ref):
    xb = x_ref[...]
    h = jnp.dot(xb, wl_ref[...], preferred_element_type=jnp.float32) + bl_ref[...]
    p = jnp.dot(xb, wr1_ref[...], preferred_element_type=jnp.float32)
    qb = jnp.dot(xb, wr2_ref[...], preferred_element_type=jnp.float32) + br_ref[...]
    h_ref[...] = h
    p_ref[...] = p
    qb_ref[...] = qb
    wa = wa_ref[...]
    sel = sel_ref[...]
    s = jnp.dot(h * wa[0] + p * wa[2], sel,
                preferred_element_type=jnp.float32)
    t = jnp.dot(h * wa[1] + qb * wa[2], sel,
                preferred_element_type=jnp.float32)
    st = jnp.concatenate([s[:, :4] + wa[3, 0], t[:, 4:8]], axis=1)
    st_ref[...] = st


def _run_prologue(x, W_l, b_l, Wr1, Wr2, b_r, wa_pack, sel):
    return pl.pallas_call(
        _prologue_body,
        grid=(N // BN,),
        in_specs=[
            pl.BlockSpec((BN, D), lambda i: (i, 0)),
            pl.BlockSpec((D, F), lambda i: (0, 0)),
            pl.BlockSpec((F,), lambda i: (0,)),
            pl.BlockSpec((D, F), lambda i: (0, 0)),
            pl.BlockSpec((D, F), lambda i: (0, 0)),
            pl.BlockSpec((F,), lambda i: (0,)),
            pl.BlockSpec((4, F), lambda i: (0, 0)),
            pl.BlockSpec((F, 8), lambda i: (0, 0)),
        ],
        out_specs=[
            pl.BlockSpec((BN, F), lambda i: (i, 0)),
            pl.BlockSpec((BN, F), lambda i: (i, 0)),
            pl.BlockSpec((BN, F), lambda i: (i, 0)),
            pl.BlockSpec((BN, 8), lambda i: (i, 0)),
        ],
        out_shape=[
            jax.ShapeDtypeStruct((N, F), jnp.float32),
            jax.ShapeDtypeStruct((N, F), jnp.float32),
            jax.ShapeDtypeStruct((N, F), jnp.float32),
            jax.ShapeDtypeStruct((N, 8), jnp.float32),
        ],
    )(x, W_l, b_l, Wr1, Wr2, b_r, wa_pack, sel)


def _zero_narrow(ref, nelem):
    iota16 = lax.iota(jnp.int32, 16)
    zval = jnp.zeros((16,), jnp.float32)

    def _z(i, carry):
        idx = iota16 + i * 16
        plsc.store_scatter(ref, [lax.shift_right_logical(idx, 3),
                                 lax.bitwise_and(idx, 7)], zval)
        return carry
    lax.fori_loop(0, nelem // 16, _z, 0)


def _make_b1_body(khalf):
    def _b1_body(src_hbm, dst_hbm, ha_hbm, hb_hbm, st_hbm,
                 acca_hbm, accb_hbm, dda_hbm, ddb_hbm,
                 tbl, srcbuf, dstbuf, rows, denbuf, zbuf, zbuf2,
                 acc_sh, dd_sh, gsem):
        cid = lax.axis_index("c")
        sid = lax.axis_index("s")
        row0 = sid * NPT

        def _zrow(r, carry):
            for cb in range(HD // 16):
                zbuf[r, pl.ds(cb * 16, 16)] = jnp.zeros((16,), jnp.float32)
            return carry
        lax.fori_loop(0, ZR, _zrow, 0)
        _zero_narrow(zbuf2, ZR * 8)
        for z in range(NPT // ZR):
            pltpu.sync_copy(zbuf, acc_sh.at[pl.ds(row0 + z * ZR, ZR)])
            pltpu.sync_copy(zbuf2, dd_sh.at[pl.ds(row0 + z * ZR, ZR)])
        pltpu.sync_copy(st_hbm, tbl)
        plsc.subcore_barrier()

        ebase = sid * EPT
        iota16 = lax.iota(jnp.int32, 16)
        _zero_narrow(denbuf, C * 8)
        if khalf == 0:
            degval = jnp.broadcast_to(jnp.where(cid == 0, 1.0, 0.0), (16,))
        else:
            degval = jnp.zeros((16,), jnp.float32)
        col1 = jnp.full((16,), 1, jnp.int32)

        def _dinit(r, carry):
            plsc.store_scatter(denbuf, [iota16 + r * 16, col1], degval)
            return carry
        lax.fori_loop(0, C // 16, _dinit, 0)

        col0 = jnp.full((16,), 0, jnp.int32)
        head = 2 * khalf + cid
        hs = jnp.full((16,), 0, jnp.int32) + head
        ht = jnp.full((16,), HEAD, jnp.int32) + head

        def _make_chunk(h_hbm):
            def _chunk(k, carry):
                base = ebase + k * C
                pltpu.sync_copy(src_hbm.at[pl.ds(base, C)], srcbuf)
                pltpu.sync_copy(dst_hbm.at[pl.ds(base, C)], dstbuf)
                pltpu.async_copy(h_hbm.at[srcbuf], rows, gsem).wait()
                for g in range(C // 16):
                    s16 = srcbuf[pl.ds(g * 16, 16)]
                    d16 = dstbuf[pl.ds(g * 16, 16)]
                    sv = plsc.load_gather(tbl, [s16, hs])
                    tv = plsc.load_gather(tbl, [d16, ht])
                    a = sv + tv
                    a = jnp.where(a > 0, a, a * 0.01)
                    ev = jnp.exp(a)
                    plsc.store_scatter(denbuf, [iota16 + g * 16, col0], ev)
                    for j in range(16):
                        row = g * 16 + j
                        w = jnp.broadcast_to(ev[j], (16,))
                        rows[row, pl.ds(0, 16)] = rows[row, pl.ds(0, 16)] * w
                        rows[row, pl.ds(16, 16)] = rows[row, pl.ds(16, 16)] * w
                pltpu.sync_copy(rows, acc_sh.at[dstbuf], add=True)
                pltpu.sync_copy(denbuf, dd_sh.at[dstbuf], add=True)
                return carry
            return _chunk

        @pl.when(cid == 0)
        def _core0():
            lax.fori_loop(0, NCHUNK, _make_chunk(ha_hbm), 0)

        @pl.when(cid == 1)
        def _core1():
            lax.fori_loop(0, NCHUNK, _make_chunk(hb_hbm), 0)

        plsc.subcore_barrier()

        @pl.when(cid == 0)
        def _dump0():
            pltpu.sync_copy(acc_sh.at[pl.ds(row0, NPT)],
                            acca_hbm.at[pl.ds(row0, NPT)])
            pltpu.sync_copy(dd_sh.at[pl.ds(row0, NPT)],
                            dda_hbm.at[pl.ds(row0, NPT)])

        @pl.when(cid == 1)
        def _dump1():
            pltpu.sync_copy(acc_sh.at[pl.ds(row0, NPT)],
                            accb_hbm.at[pl.ds(row0, NPT)])
            pltpu.sync_copy(dd_sh.at[pl.ds(row0, NPT)],
                            ddb_hbm.at[pl.ds(row0, NPT)])

    return _b1_body


def _run_b1(khalf, src, dst, h_a, h_b, st):
    f = pl.kernel(
        _make_b1_body(khalf),
        out_type=[
            jax.ShapeDtypeStruct((NPAD, HD), jnp.float32),
            jax.ShapeDtypeStruct((NPAD, HD), jnp.float32),
            jax.ShapeDtypeStruct((NPAD, 8), jnp.float32),
            jax.ShapeDtypeStruct((NPAD, 8), jnp.float32),
        ],
        mesh=_sc_mesh(),
        compiler_params=_SC_PARAMS,
        scratch_types=[
            pltpu.VMEM((N, 8), jnp.float32),
            pltpu.VMEM((C,), jnp.int32),
            pltpu.VMEM((C,), jnp.int32),
            pltpu.VMEM((C, HD), jnp.float32),
            pltpu.VMEM((C, 8), jnp.float32),
            pltpu.VMEM((ZR, HD), jnp.float32),
            pltpu.VMEM((ZR, 8), jnp.float32),
            pltpu.VMEM_SHARED((NPAD, HD), jnp.float32),
            pltpu.VMEM_SHARED((NPAD, 8), jnp.float32),
            pltpu.SemaphoreType.DMA,
        ],
    )
    return f(src, dst, h_a, h_b, st)


def _b2_body(src_hbm, dst_hbm, plo_hbm, phi_hbm,
             acclo_hbm, acchi_hbm,
             srcbuf, dstbuf, rows, zbuf, acc_sh, gsem):
    cid = lax.axis_index("c")
    sid = lax.axis_index("s")
    row0 = sid * NPT

    def _zrow(r, carry):
        for cb in range(FH // 16):
            zbuf[r, pl.ds(cb * 16, 16)] = jnp.zeros((16,), jnp.float32)
        return carry
    lax.fori_loop(0, ZR, _zrow, 0)
    for z in range(NPT // ZR):
        pltpu.sync_copy(zbuf, acc_sh.at[pl.ds(row0 + z * ZR, ZR)])
    plsc.subcore_barrier()

    ebase = sid * EPT

    def _make_chunk(p_hbm):
        def _chunk(k, carry):
            base = ebase + k * C2
            pltpu.sync_copy(src_hbm.at[pl.ds(base, C2)], srcbuf)
            pltpu.sync_copy(dst_hbm.at[pl.ds(base, C2)], dstbuf)
            pltpu.async_copy(p_hbm.at[srcbuf], rows, gsem).wait()
            pltpu.sync_copy(rows, acc_sh.at[dstbuf], add=True)
            return carry
        return _chunk

    @pl.when(cid == 0)
    def _core0():
        lax.fori_loop(0, NCHUNK2, _make_chunk(plo_hbm), 0)

    @pl.when(cid == 1)
    def _core1():
        lax.fori_loop(0, NCHUNK2, _make_chunk(phi_hbm), 0)

    plsc.subcore_barrier()

    @pl.when(cid == 0)
    def _dump0():
        pltpu.sync_copy(acc_sh.at[pl.ds(row0, NPT)],
                        acclo_hbm.at[pl.ds(row0, NPT)])

    @pl.when(cid == 1)
    def _dump1():
        pltpu.sync_copy(acc_sh.at[pl.ds(row0, NPT)],
                        acchi_hbm.at[pl.ds(row0, NPT)])


def _run_b2(src, dst, p_lo, p_hi):
    f = pl.kernel(
        _b2_body,
        out_type=[
            jax.ShapeDtypeStruct((NPAD, FH), jnp.float32),
            jax.ShapeDtypeStruct((NPAD, FH), jnp.float32),
        ],
        mesh=_sc_mesh(),
        compiler_params=_SC_PARAMS,
        scratch_types=[
            pltpu.VMEM((C2,), jnp.int32),
            pltpu.VMEM((C2,), jnp.int32),
            pltpu.VMEM((C2, FH), jnp.float32),
            pltpu.VMEM((ZR, FH), jnp.float32),
            pltpu.VMEM_SHARED((NPAD, FH), jnp.float32),
            pltpu.SemaphoreType.DMA,
        ],
    )
    return f(src, dst, p_lo, p_hi)


def _epilogue_body(a0_ref, a1_ref, a2_ref, a3_ref,
                   d0_ref, d1_ref, d2_ref, d3_ref,
                   eal_ref, eah_ref, qb_ref, selT_ref, out_ref, es_ref):
    den4 = jnp.concatenate([d0_ref[...][:, 0:1], d1_ref[...][:, 0:1],
                            d2_ref[...][:, 0:1], d3_ref[...][:, 0:1]], axis=1)
    den128 = jnp.dot(den4, selT_ref[...], preferred_element_type=jnp.float32)
    deg128 = jnp.broadcast_to(d0_ref[...][:, 1:2], (BN, F))
    den_safe = jnp.where(den128 > 0, den128, 1.0)
    oa = jnp.concatenate([a0_ref[...], a1_ref[...], a2_ref[...], a3_ref[...]],
                         axis=1)
    ea = jnp.concatenate([eal_ref[...], eah_ref[...]], axis=1)
    out_ref[...] = oa / den_safe
    es_ref[...] = (ea + deg128 * qb_ref[...]) / jnp.maximum(deg128, 1.0)


def _run_epilogue(accs, dds, eal, eah, qb, selT):
    return pl.pallas_call(
        _epilogue_body,
        grid=(N // BN,),
        in_specs=[
            pl.BlockSpec((BN, HD), lambda i: (i, 0)),
            pl.BlockSpec((BN, HD), lambda i: (i, 0)),
            pl.BlockSpec((BN, HD), lambda i: (i, 0)),
            pl.BlockSpec((BN, HD), lambda i: (i, 0)),
            pl.BlockSpec((BN, 8), lambda i: (i, 0)),
            pl.BlockSpec((BN, 8), lambda i: (i, 0)),
            pl.BlockSpec((BN, 8), lambda i: (i, 0)),
            pl.BlockSpec((BN, 8), lambda i: (i, 0)),
            pl.BlockSpec((BN, FH), lambda i: (i, 0)),
            pl.BlockSpec((BN, FH), lambda i: (i, 0)),
            pl.BlockSpec((BN, F), lambda i: (i, 0)),
            pl.BlockSpec((4, F), lambda i: (0, 0)),
        ],
        out_specs=[
            pl.BlockSpec((BN, F), lambda i: (i, 0)),
            pl.BlockSpec((BN, F), lambda i: (i, 0)),
        ],
        out_shape=[
            jax.ShapeDtypeStruct((N, F), jnp.float32),
            jax.ShapeDtypeStruct((N, F), jnp.float32),
        ],
    )(*accs, *dds, eal, eah, qb, selT)


@jax.jit
def kernel(x, edge_index, W_r, b_r, W_l, b_l, W_a, b_a):
    src = edge_index[0]
    dst = edge_index[1]
    Wr1 = W_r[:D]
    Wr2 = W_r[D:]
    w1 = jnp.tile(W_a[0:HD, 0], HEAD)
    w2 = jnp.tile(W_a[HD:2 * HD, 0], HEAD)
    w3 = jnp.tile(W_a[2 * HD:, 0], HEAD)
    ba_row = jnp.zeros((F,), jnp.float32).at[0].set(b_a[0])
    wa_pack = jnp.stack([w1, w2, w3, ba_row], axis=0)
    didx = jnp.arange(F) // HD
    sel = jnp.stack([(didx == (hh % HEAD)).astype(jnp.float32)
                     for hh in range(8)], axis=1)
    selT = jnp.stack([(didx == hh).astype(jnp.float32)
                      for hh in range(HEAD)], axis=0)

    h, p, qb, st = _run_prologue(x, W_l, b_l, Wr1, Wr2, b_r, wa_pack, sel)
    h0, h1, h2, h3 = (h[:, i * HD:(i + 1) * HD] for i in range(HEAD))
    p_lo = p[:, :FH]
    p_hi = p[:, FH:]
    a0, a1, dd0, dd1 = _run_b1(0, src, dst, h0, h1, st)
    a2, a3, dd2, dd3 = _run_b1(1, src, dst, h2, h3, st)
    eal, eah = _run_b2(src, dst, p_lo, p_hi)
    out, es = _run_epilogue(
        [a0[:N], a1[:N], a2[:N], a3[:N]],
        [dd0[:N], dd1[:N], dd2[:N], dd3[:N]],
        eal[:N], eah[:N], qb, selT)
    return (es, out, h)


# speedup vs baseline: 101.8240x; 1.3534x over previous
import jax
import jax.numpy as jnp
from jax import lax
from jax.experimental import pallas as pl
from jax.experimental.pallas import tpu as pltpu
from jax.experimental.pallas import tpu_sc as plsc

N = 10000
E = 320000
D = 128
HEAD = 4
HD = 32
F = HEAD * HD
FH = F // 2

NC = 2
NS = 16

EPT = E // NS
C = 160
NCHUNK = EPT // C
C2 = 400
NCHUNK2 = EPT // C2
NPAD = 10240
NPT = NPAD // NS
ZR = 128

BN = 1000

_SC_PARAMS = pltpu.CompilerParams(needs_layout_passes=False,
                                  use_tc_tiling_on_sc=False)


def _sc_mesh():
    return plsc.VectorSubcoreMesh(core_axis_name="c", subcore_axis_name="s",
                                  num_cores=NC, num_subcores=NS)


def _prologue_body(x_ref, wl_ref, bl_ref, wr1_ref, wr2_ref, br_ref, wa_ref,
                   sel_ref, h_ref, p_ref, qb_ref, st_ref):
    xb = x_ref[...]
    h = jnp.dot(xb, wl_ref[...], preferred_element_type=jnp.float32) + bl_ref[...]
    p = jnp.dot(xb, wr1_ref[...], preferred_element_type=jnp.float32)
    qb = jnp.dot(xb, wr2_ref[...], preferred_element_type=jnp.float32) + br_ref[...]
    h_ref[...] = h
    p_ref[...] = p
    qb_ref[...] = qb
    wa = wa_ref[...]
    sel = sel_ref[...]
    s = jnp.dot(h * wa[0] + p * wa[2], sel,
                preferred_element_type=jnp.float32)
    t = jnp.dot(h * wa[1] + qb * wa[2], sel,
                preferred_element_type=jnp.float32)
    st = jnp.concatenate([s[:, :4] + wa[3, 0], t[:, 4:8]], axis=1)
    st_ref[...] = st


def _run_prologue(x, W_l, b_l, Wr1, Wr2, b_r, wa_pack, sel):
    return pl.pallas_call(
        _prologue_body,
        grid=(N // BN,),
        in_specs=[
            pl.BlockSpec((BN, D), lambda i: (i, 0)),
            pl.BlockSpec((D, F), lambda i: (0, 0)),
            pl.BlockSpec((F,), lambda i: (0,)),
            pl.BlockSpec((D, F), lambda i: (0, 0)),
            pl.BlockSpec((D, F), lambda i: (0, 0)),
            pl.BlockSpec((F,), lambda i: (0,)),
            pl.BlockSpec((4, F), lambda i: (0, 0)),
            pl.BlockSpec((F, 8), lambda i: (0, 0)),
        ],
        out_specs=[
            pl.BlockSpec((BN, F), lambda i: (i, 0)),
            pl.BlockSpec((BN, F), lambda i: (i, 0)),
            pl.BlockSpec((BN, F), lambda i: (i, 0)),
            pl.BlockSpec((BN, 8), lambda i: (i, 0)),
        ],
        out_shape=[
            jax.ShapeDtypeStruct((N, F), jnp.float32),
            jax.ShapeDtypeStruct((N, F), jnp.float32),
            jax.ShapeDtypeStruct((N, F), jnp.float32),
            jax.ShapeDtypeStruct((N, 8), jnp.float32),
        ],
    )(x, W_l, b_l, Wr1, Wr2, b_r, wa_pack, sel)


def _zero_narrow(ref, nelem):
    iota16 = lax.iota(jnp.int32, 16)
    zval = jnp.zeros((16,), jnp.float32)

    def _z(i, carry):
        idx = iota16 + i * 16
        plsc.store_scatter(ref, [lax.shift_right_logical(idx, 3),
                                 lax.bitwise_and(idx, 7)], zval)
        return carry
    lax.fori_loop(0, nelem // 16, _z, 0)


def _make_b1_body(khalf):
    def _b1_body(src_hbm, dst_hbm, ha_hbm, hb_hbm, st_hbm,
                 acca_hbm, accb_hbm, dda_hbm, ddb_hbm,
                 tbl, src0, src1, dst0, dst1, rows0, rows1, den0, den1,
                 zbuf, zbuf2, acc_sh, dd_sh, gsem0, gsem1):
        cid = lax.axis_index("c")
        sid = lax.axis_index("s")
        row0 = sid * NPT
        srcb = (src0, src1)
        dstb = (dst0, dst1)
        rowsb = (rows0, rows1)
        denb = (den0, den1)
        gsem = (gsem0, gsem1)

        def _zrow(r, carry):
            for cb in range(HD // 16):
                zbuf[r, pl.ds(cb * 16, 16)] = jnp.zeros((16,), jnp.float32)
            return carry
        lax.fori_loop(0, ZR, _zrow, 0)
        _zero_narrow(zbuf2, ZR * 8)
        for z in range(NPT // ZR):
            pltpu.sync_copy(zbuf, acc_sh.at[pl.ds(row0 + z * ZR, ZR)])
            pltpu.sync_copy(zbuf2, dd_sh.at[pl.ds(row0 + z * ZR, ZR)])
        pltpu.sync_copy(st_hbm, tbl)
        plsc.subcore_barrier()

        ebase = sid * EPT
        iota16 = lax.iota(jnp.int32, 16)
        if khalf == 0:
            degval = jnp.broadcast_to(jnp.where(cid == 0, 1.0, 0.0), (16,))
        else:
            degval = jnp.zeros((16,), jnp.float32)
        col1 = jnp.full((16,), 1, jnp.int32)
        for b in range(2):
            _zero_narrow(denb[b], C * 8)

            def _dinit(r, carry, _d=denb[b]):
                plsc.store_scatter(_d, [iota16 + r * 16, col1], degval)
                return carry
            lax.fori_loop(0, C // 16, _dinit, 0)

        col0 = jnp.full((16,), 0, jnp.int32)
        head = 2 * khalf + cid
        hs = jnp.full((16,), 0, jnp.int32) + head
        ht = jnp.full((16,), HEAD, jnp.int32) + head

        def _compute(b):
            srcq, dstq, rows, denbuf = srcb[b], dstb[b], rowsb[b], denb[b]
            for g in range(C // 16):
                s16 = srcq[pl.ds(g * 16, 16)]
                d16 = dstq[pl.ds(g * 16, 16)]
                sv = plsc.load_gather(tbl, [s16, hs])
                tv = plsc.load_gather(tbl, [d16, ht])
                a = sv + tv
                a = jnp.where(a > 0, a, a * 0.01)
                ev = jnp.exp(a)
                plsc.store_scatter(denbuf, [iota16 + g * 16, col0], ev)
                for j in range(16):
                    row = g * 16 + j
                    w = jnp.broadcast_to(ev[j], (16,))
                    rows[row, pl.ds(0, 16)] = rows[row, pl.ds(0, 16)] * w
                    rows[row, pl.ds(16, 16)] = rows[row, pl.ds(16, 16)] * w
            pltpu.sync_copy(rows, acc_sh.at[dstq], add=True)
            pltpu.sync_copy(denbuf, dd_sh.at[dstq], add=True)

        def _fetch(h_hbm, c, b):
            base = ebase + c * C
            pltpu.sync_copy(src_hbm.at[pl.ds(base, C)], srcb[b])
            pltpu.sync_copy(dst_hbm.at[pl.ds(base, C)], dstb[b])
            pltpu.async_copy(h_hbm.at[srcb[b]], rowsb[b], gsem[b])

        def _run_edges(h_hbm):
            _fetch(h_hbm, 0, 0)

            def _pair(kk, carry):
                for b in range(2):
                    c = 2 * kk + b
                    _fetch(h_hbm, c + 1, 1 - b)
                    pltpu.make_async_copy(h_hbm.at[srcb[b]], rowsb[b],
                                          gsem[b]).wait()
                    _compute(b)
                return carry
            lax.fori_loop(0, NCHUNK // 2, _pair, 0)
            pltpu.make_async_copy(h_hbm.at[srcb[0]], rowsb[0], gsem[0]).wait()
            _compute(0)

        @pl.when(cid == 0)
        def _core0():
            _run_edges(ha_hbm)

        @pl.when(cid == 1)
        def _core1():
            _run_edges(hb_hbm)

        plsc.subcore_barrier()

        @pl.when(cid == 0)
        def _dump0():
            pltpu.sync_copy(acc_sh.at[pl.ds(row0, NPT)],
                            acca_hbm.at[pl.ds(row0, NPT)])
            pltpu.sync_copy(dd_sh.at[pl.ds(row0, NPT)],
                            dda_hbm.at[pl.ds(row0, NPT)])

        @pl.when(cid == 1)
        def _dump1():
            pltpu.sync_copy(acc_sh.at[pl.ds(row0, NPT)],
                            accb_hbm.at[pl.ds(row0, NPT)])
            pltpu.sync_copy(dd_sh.at[pl.ds(row0, NPT)],
                            ddb_hbm.at[pl.ds(row0, NPT)])

    return _b1_body


def _run_b1(khalf, src, dst, h_a, h_b, st):
    f = pl.kernel(
        _make_b1_body(khalf),
        out_type=[
            jax.ShapeDtypeStruct((NPAD, HD), jnp.float32),
            jax.ShapeDtypeStruct((NPAD, HD), jnp.float32),
            jax.ShapeDtypeStruct((NPAD, 8), jnp.float32),
            jax.ShapeDtypeStruct((NPAD, 8), jnp.float32),
        ],
        mesh=_sc_mesh(),
        compiler_params=_SC_PARAMS,
        scratch_types=[
            pltpu.VMEM((N, 8), jnp.float32),
            pltpu.VMEM((C,), jnp.int32),
            pltpu.VMEM((C,), jnp.int32),
            pltpu.VMEM((C,), jnp.int32),
            pltpu.VMEM((C,), jnp.int32),
            pltpu.VMEM((C, HD), jnp.float32),
            pltpu.VMEM((C, HD), jnp.float32),
            pltpu.VMEM((C, 8), jnp.float32),
            pltpu.VMEM((C, 8), jnp.float32),
            pltpu.VMEM((ZR, HD), jnp.float32),
            pltpu.VMEM((ZR, 8), jnp.float32),
            pltpu.VMEM_SHARED((NPAD, HD), jnp.float32),
            pltpu.VMEM_SHARED((NPAD, 8), jnp.float32),
            pltpu.SemaphoreType.DMA,
            pltpu.SemaphoreType.DMA,
        ],
    )
    return f(src, dst, h_a, h_b, st)


def _b2_body(src_hbm, dst_hbm, plo_hbm, phi_hbm,
             acclo_hbm, acchi_hbm,
             src0, src1, dst0, dst1, rows0, rows1, zbuf, acc_sh,
             gsem0, gsem1):
    cid = lax.axis_index("c")
    sid = lax.axis_index("s")
    row0 = sid * NPT
    srcb = (src0, src1)
    dstb = (dst0, dst1)
    rowsb = (rows0, rows1)
    gsem = (gsem0, gsem1)

    def _zrow(r, carry):
        for cb in range(FH // 16):
            zbuf[r, pl.ds(cb * 16, 16)] = jnp.zeros((16,), jnp.float32)
        return carry
    lax.fori_loop(0, ZR, _zrow, 0)
    for z in range(NPT // ZR):
        pltpu.sync_copy(zbuf, acc_sh.at[pl.ds(row0 + z * ZR, ZR)])
    plsc.subcore_barrier()

    ebase = sid * EPT

    def _fetch(p_hbm, c, b):
        base = ebase + c * C2
        pltpu.sync_copy(src_hbm.at[pl.ds(base, C2)], srcb[b])
        pltpu.sync_copy(dst_hbm.at[pl.ds(base, C2)], dstb[b])
        pltpu.async_copy(p_hbm.at[srcb[b]], rowsb[b], gsem[b])

    def _drain_scatter(p_hbm, b):
        pltpu.make_async_copy(p_hbm.at[srcb[b]], rowsb[b], gsem[b]).wait()
        pltpu.sync_copy(rowsb[b], acc_sh.at[dstb[b]], add=True)

    def _run_edges(p_hbm):
        _fetch(p_hbm, 0, 0)

        def _pair(kk, carry):
            for b in range(2):
                c = 2 * kk + b
                _fetch(p_hbm, c + 1, 1 - b)
                _drain_scatter(p_hbm, b)
            return carry
        lax.fori_loop(0, NCHUNK2 // 2 - 1, _pair, 0)
        _fetch(p_hbm, NCHUNK2 - 1, 1)
        _drain_scatter(p_hbm, 0)
        _drain_scatter(p_hbm, 1)

    @pl.when(cid == 0)
    def _core0():
        _run_edges(plo_hbm)

    @pl.when(cid == 1)
    def _core1():
        _run_edges(phi_hbm)

    plsc.subcore_barrier()

    @pl.when(cid == 0)
    def _dump0():
        pltpu.sync_copy(acc_sh.at[pl.ds(row0, NPT)],
                        acclo_hbm.at[pl.ds(row0, NPT)])

    @pl.when(cid == 1)
    def _dump1():
        pltpu.sync_copy(acc_sh.at[pl.ds(row0, NPT)],
                        acchi_hbm.at[pl.ds(row0, NPT)])


def _run_b2(src, dst, p_lo, p_hi):
    f = pl.kernel(
        _b2_body,
        out_type=[
            jax.ShapeDtypeStruct((NPAD, FH), jnp.float32),
            jax.ShapeDtypeStruct((NPAD, FH), jnp.float32),
        ],
        mesh=_sc_mesh(),
        compiler_params=_SC_PARAMS,
        scratch_types=[
            pltpu.VMEM((C2,), jnp.int32),
            pltpu.VMEM((C2,), jnp.int32),
            pltpu.VMEM((C2,), jnp.int32),
            pltpu.VMEM((C2,), jnp.int32),
            pltpu.VMEM((C2, FH), jnp.float32),
            pltpu.VMEM((C2, FH), jnp.float32),
            pltpu.VMEM((ZR, FH), jnp.float32),
            pltpu.VMEM_SHARED((NPAD, FH), jnp.float32),
            pltpu.SemaphoreType.DMA,
            pltpu.SemaphoreType.DMA,
        ],
    )
    return f(src, dst, p_lo, p_hi)


def _epilogue_body(a0_ref, a1_ref, a2_ref, a3_ref,
                   d0_ref, d1_ref, d2_ref, d3_ref,
                   eal_ref, eah_ref, qb_ref, selT_ref, out_ref, es_ref):
    den4 = jnp.concatenate([d0_ref[...][:, 0:1], d1_ref[...][:, 0:1],
                            d2_ref[...][:, 0:1], d3_ref[...][:, 0:1]], axis=1)
    den128 = jnp.dot(den4, selT_ref[...], preferred_element_type=jnp.float32)
    deg128 = jnp.broadcast_to(d0_ref[...][:, 1:2], (BN, F))
    den_safe = jnp.where(den128 > 0, den128, 1.0)
    oa = jnp.concatenate([a0_ref[...], a1_ref[...], a2_ref[...], a3_ref[...]],
                         axis=1)
    ea = jnp.concatenate([eal_ref[...], eah_ref[...]], axis=1)
    out_ref[...] = oa / den_safe
    es_ref[...] = (ea + deg128 * qb_ref[...]) / jnp.maximum(deg128, 1.0)


def _run_epilogue(accs, dds, eal, eah, qb, selT):
    return pl.pallas_call(
        _epilogue_body,
        grid=(N // BN,),
        in_specs=[
            pl.BlockSpec((BN, HD), lambda i: (i, 0)),
            pl.BlockSpec((BN, HD), lambda i: (i, 0)),
            pl.BlockSpec((BN, HD), lambda i: (i, 0)),
            pl.BlockSpec((BN, HD), lambda i: (i, 0)),
            pl.BlockSpec((BN, 8), lambda i: (i, 0)),
            pl.BlockSpec((BN, 8), lambda i: (i, 0)),
            pl.BlockSpec((BN, 8), lambda i: (i, 0)),
            pl.BlockSpec((BN, 8), lambda i: (i, 0)),
            pl.BlockSpec((BN, FH), lambda i: (i, 0)),
            pl.BlockSpec((BN, FH), lambda i: (i, 0)),
            pl.BlockSpec((BN, F), lambda i: (i, 0)),
            pl.BlockSpec((4, F), lambda i: (0, 0)),
        ],
        out_specs=[
            pl.BlockSpec((BN, F), lambda i: (i, 0)),
            pl.BlockSpec((BN, F), lambda i: (i, 0)),
        ],
        out_shape=[
            jax.ShapeDtypeStruct((N, F), jnp.float32),
            jax.ShapeDtypeStruct((N, F), jnp.float32),
        ],
    )(*accs, *dds, eal, eah, qb, selT)


@jax.jit
def kernel(x, edge_index, W_r, b_r, W_l, b_l, W_a, b_a):
    src = edge_index[0]
    dst = edge_index[1]
    Wr1 = W_r[:D]
    Wr2 = W_r[D:]
    w1 = jnp.tile(W_a[0:HD, 0], HEAD)
    w2 = jnp.tile(W_a[HD:2 * HD, 0], HEAD)
    w3 = jnp.tile(W_a[2 * HD:, 0], HEAD)
    ba_row = jnp.zeros((F,), jnp.float32).at[0].set(b_a[0])
    wa_pack = jnp.stack([w1, w2, w3, ba_row], axis=0)
    didx = jnp.arange(F) // HD
    sel = jnp.stack([(didx == (hh % HEAD)).astype(jnp.float32)
                     for hh in range(8)], axis=1)
    selT = jnp.stack([(didx == hh).astype(jnp.float32)
                      for hh in range(HEAD)], axis=0)

    h, p, qb, st = _run_prologue(x, W_l, b_l, Wr1, Wr2, b_r, wa_pack, sel)
    h0, h1, h2, h3 = (h[:, i * HD:(i + 1) * HD] for i in range(HEAD))
    p_lo = p[:, :FH]
    p_hi = p[:, FH:]
    a0, a1, dd0, dd1 = _run_b1(0, src, dst, h0, h1, st)
    a2, a3, dd2, dd3 = _run_b1(1, src, dst, h2, h3, st)
    eal, eah = _run_b2(src, dst, p_lo, p_hi)
    out, es = _run_epilogue(
        [a0[:N], a1[:N], a2[:N], a3[:N]],
        [dd0[:N], dd1[:N], dd2[:N], dd3[:N]],
        eal[:N], eah[:N], qb, selT)
    return (es, out, h)


# speedup vs baseline: 109.8525x; 1.0788x over previous
import jax
import jax.numpy as jnp
from jax import lax
from jax.experimental import pallas as pl
from jax.experimental.pallas import tpu as pltpu
from jax.experimental.pallas import tpu_sc as plsc

N = 10000
E = 320000
D = 128
HEAD = 4
HD = 32
F = HEAD * HD
FH = F // 2

NC = 2
NS = 16

EPT = E // NS
C = 160
NCHUNK = EPT // C
C2 = 400
NCHUNK2 = EPT // C2
NPAD = 10240
NPT = NPAD // NS
ZR = 128

BN = 1000

_SC_PARAMS = pltpu.CompilerParams(needs_layout_passes=False,
                                  use_tc_tiling_on_sc=False)


def _sc_mesh():
    return plsc.VectorSubcoreMesh(core_axis_name="c", subcore_axis_name="s",
                                  num_cores=NC, num_subcores=NS)


def _prologue_body(x_ref, wl_ref, bl_ref, wr1_ref, wr2_ref, br_ref, wa_ref,
                   sel_ref, h_ref, p_ref, qb_ref, st_ref):
    xb = x_ref[...]
    h = jnp.dot(xb, wl_ref[...], preferred_element_type=jnp.float32) + bl_ref[...]
    p = jnp.dot(xb, wr1_ref[...], preferred_element_type=jnp.float32)
    qb = jnp.dot(xb, wr2_ref[...], preferred_element_type=jnp.float32) + br_ref[...]
    h_ref[...] = h
    p_ref[...] = p
    qb_ref[...] = qb
    wa = wa_ref[...]
    sel = sel_ref[...]
    s = jnp.dot(h * wa[0] + p * wa[2], sel,
                preferred_element_type=jnp.float32)
    t = jnp.dot(h * wa[1] + qb * wa[2], sel,
                preferred_element_type=jnp.float32)
    st = jnp.concatenate([s[:, :4] + wa[3, 0], t[:, 4:8]], axis=1)
    st_ref[...] = st


def _run_prologue(x, W_l, b_l, Wr1, Wr2, b_r, wa_pack, sel):
    return pl.pallas_call(
        _prologue_body,
        grid=(N // BN,),
        in_specs=[
            pl.BlockSpec((BN, D), lambda i: (i, 0)),
            pl.BlockSpec((D, F), lambda i: (0, 0)),
            pl.BlockSpec((F,), lambda i: (0,)),
            pl.BlockSpec((D, F), lambda i: (0, 0)),
            pl.BlockSpec((D, F), lambda i: (0, 0)),
            pl.BlockSpec((F,), lambda i: (0,)),
            pl.BlockSpec((4, F), lambda i: (0, 0)),
            pl.BlockSpec((F, 8), lambda i: (0, 0)),
        ],
        out_specs=[
            pl.BlockSpec((BN, F), lambda i: (i, 0)),
            pl.BlockSpec((BN, F), lambda i: (i, 0)),
            pl.BlockSpec((BN, F), lambda i: (i, 0)),
            pl.BlockSpec((BN, 8), lambda i: (i, 0)),
        ],
        out_shape=[
            jax.ShapeDtypeStruct((N, F), jnp.float32),
            jax.ShapeDtypeStruct((N, F), jnp.float32),
            jax.ShapeDtypeStruct((N, F), jnp.float32),
            jax.ShapeDtypeStruct((N, 8), jnp.float32),
        ],
    )(x, W_l, b_l, Wr1, Wr2, b_r, wa_pack, sel)


def _zero_narrow(ref, nelem):
    iota16 = lax.iota(jnp.int32, 16)
    zval = jnp.zeros((16,), jnp.float32)

    def _z(i, carry):
        idx = iota16 + i * 16
        plsc.store_scatter(ref, [lax.shift_right_logical(idx, 3),
                                 lax.bitwise_and(idx, 7)], zval)
        return carry
    lax.fori_loop(0, nelem // 16, _z, 0)


def _make_b1_body(khalf):
    def _b1_body(src_hbm, dst_hbm, ha_hbm, hb_hbm, st_hbm,
                 acca_hbm, accb_hbm, dda_hbm, ddb_hbm,
                 tbl, src0, src1, dst0, dst1, rows0, rows1, den0, den1,
                 zbuf, zbuf2, acc_sh, dd_sh, gsem0, gsem1, ssem0, ssem1):
        cid = lax.axis_index("c")
        sid = lax.axis_index("s")
        row0 = sid * NPT
        srcb = (src0, src1)
        dstb = (dst0, dst1)
        rowsb = (rows0, rows1)
        denb = (den0, den1)
        gsem = (gsem0, gsem1)
        ssemb = (ssem0, ssem1)

        def _zrow(r, carry):
            for cb in range(HD // 16):
                zbuf[r, pl.ds(cb * 16, 16)] = jnp.zeros((16,), jnp.float32)
            return carry
        lax.fori_loop(0, ZR, _zrow, 0)
        _zero_narrow(zbuf2, ZR * 8)
        for z in range(NPT // ZR):
            pltpu.sync_copy(zbuf, acc_sh.at[pl.ds(row0 + z * ZR, ZR)])
            pltpu.sync_copy(zbuf2, dd_sh.at[pl.ds(row0 + z * ZR, ZR)])
        pltpu.sync_copy(st_hbm, tbl)
        plsc.subcore_barrier()

        ebase = sid * EPT
        iota16 = lax.iota(jnp.int32, 16)
        if khalf == 0:
            degval = jnp.broadcast_to(jnp.where(cid == 0, 1.0, 0.0), (16,))
        else:
            degval = jnp.zeros((16,), jnp.float32)
        col1 = jnp.full((16,), 1, jnp.int32)
        for b in range(2):
            _zero_narrow(denb[b], C * 8)

            def _dinit(r, carry, _d=denb[b]):
                plsc.store_scatter(_d, [iota16 + r * 16, col1], degval)
                return carry
            lax.fori_loop(0, C // 16, _dinit, 0)

        col0 = jnp.full((16,), 0, jnp.int32)
        head = 2 * khalf + cid
        hs = jnp.full((16,), 0, jnp.int32) + head
        ht = jnp.full((16,), HEAD, jnp.int32) + head

        def _compute(b):
            srcq, dstq, rows, denbuf = srcb[b], dstb[b], rowsb[b], denb[b]
            for g in range(C // 16):
                s16 = srcq[pl.ds(g * 16, 16)]
                d16 = dstq[pl.ds(g * 16, 16)]
                sv = plsc.load_gather(tbl, [s16, hs])
                tv = plsc.load_gather(tbl, [d16, ht])
                a = sv + tv
                a = jnp.where(a > 0, a, a * 0.01)
                ev = jnp.exp(a)
                plsc.store_scatter(denbuf, [iota16 + g * 16, col0], ev)
                for j in range(16):
                    row = g * 16 + j
                    w = jnp.broadcast_to(ev[j], (16,))
                    rows[row, pl.ds(0, 16)] = rows[row, pl.ds(0, 16)] * w
                    rows[row, pl.ds(16, 16)] = rows[row, pl.ds(16, 16)] * w
                pltpu.async_copy(rows.at[pl.ds(g * 16, 16)], acc_sh.at[d16],
                                 ssemb[b], add=True)
                pltpu.async_copy(denbuf.at[pl.ds(g * 16, 16)], dd_sh.at[d16],
                                 ssemb[b], add=True)

        def _wait_scatter(b):
            pltpu.make_async_copy(rowsb[b], acc_sh.at[dstb[b]],
                                  ssemb[b]).wait()
            pltpu.make_async_copy(denb[b], dd_sh.at[dstb[b]], ssemb[b]).wait()

        def _fetch(h_hbm, c, b):
            base = ebase + c * C
            pltpu.sync_copy(src_hbm.at[pl.ds(base, C)], srcb[b])
            pltpu.sync_copy(dst_hbm.at[pl.ds(base, C)], dstb[b])
            pltpu.async_copy(h_hbm.at[srcb[b]], rowsb[b], gsem[b])

        def _run_edges(h_hbm):
            _fetch(h_hbm, 0, 0)

            def _pair(kk, carry):
                for b in range(2):
                    c = 2 * kk + b
                    if b == 0:
                        @pl.when(kk > 0)
                        def _():
                            _wait_scatter(1)
                    else:
                        _wait_scatter(0)
                    _fetch(h_hbm, c + 1, 1 - b)
                    pltpu.make_async_copy(h_hbm.at[srcb[b]], rowsb[b],
                                          gsem[b]).wait()
                    _compute(b)
                return carry
            lax.fori_loop(0, NCHUNK // 2, _pair, 0)
            pltpu.make_async_copy(h_hbm.at[srcb[0]], rowsb[0], gsem[0]).wait()
            _compute(0)
            _wait_scatter(1)
            _wait_scatter(0)

        @pl.when(cid == 0)
        def _core0():
            _run_edges(ha_hbm)

        @pl.when(cid == 1)
        def _core1():
            _run_edges(hb_hbm)

        plsc.subcore_barrier()

        @pl.when(cid == 0)
        def _dump0():
            pltpu.sync_copy(acc_sh.at[pl.ds(row0, NPT)],
                            acca_hbm.at[pl.ds(row0, NPT)])
            pltpu.sync_copy(dd_sh.at[pl.ds(row0, NPT)],
                            dda_hbm.at[pl.ds(row0, NPT)])

        @pl.when(cid == 1)
        def _dump1():
            pltpu.sync_copy(acc_sh.at[pl.ds(row0, NPT)],
                            accb_hbm.at[pl.ds(row0, NPT)])
            pltpu.sync_copy(dd_sh.at[pl.ds(row0, NPT)],
                            ddb_hbm.at[pl.ds(row0, NPT)])

    return _b1_body


def _run_b1(khalf, src, dst, h_a, h_b, st):
    f = pl.kernel(
        _make_b1_body(khalf),
        out_type=[
            jax.ShapeDtypeStruct((NPAD, HD), jnp.float32),
            jax.ShapeDtypeStruct((NPAD, HD), jnp.float32),
            jax.ShapeDtypeStruct((NPAD, 8), jnp.float32),
            jax.ShapeDtypeStruct((NPAD, 8), jnp.float32),
        ],
        mesh=_sc_mesh(),
        compiler_params=_SC_PARAMS,
        scratch_types=[
            pltpu.VMEM((N, 8), jnp.float32),
            pltpu.VMEM((C,), jnp.int32),
            pltpu.VMEM((C,), jnp.int32),
            pltpu.VMEM((C,), jnp.int32),
            pltpu.VMEM((C,), jnp.int32),
            pltpu.VMEM((C, HD), jnp.float32),
            pltpu.VMEM((C, HD), jnp.float32),
            pltpu.VMEM((C, 8), jnp.float32),
            pltpu.VMEM((C, 8), jnp.float32),
            pltpu.VMEM((ZR, HD), jnp.float32),
            pltpu.VMEM((ZR, 8), jnp.float32),
            pltpu.VMEM_SHARED((NPAD, HD), jnp.float32),
            pltpu.VMEM_SHARED((NPAD, 8), jnp.float32),
            pltpu.SemaphoreType.DMA,
            pltpu.SemaphoreType.DMA,
            pltpu.SemaphoreType.DMA,
            pltpu.SemaphoreType.DMA,
        ],
    )
    return f(src, dst, h_a, h_b, st)


def _b2_body(src_hbm, dst_hbm, plo_hbm, phi_hbm,
             acclo_hbm, acchi_hbm,
             src0, src1, dst0, dst1, rows0, rows1, zbuf, acc_sh,
             gsem0, gsem1):
    cid = lax.axis_index("c")
    sid = lax.axis_index("s")
    row0 = sid * NPT
    srcb = (src0, src1)
    dstb = (dst0, dst1)
    rowsb = (rows0, rows1)
    gsem = (gsem0, gsem1)

    def _zrow(r, carry):
        for cb in range(FH // 16):
            zbuf[r, pl.ds(cb * 16, 16)] = jnp.zeros((16,), jnp.float32)
        return carry
    lax.fori_loop(0, ZR, _zrow, 0)
    for z in range(NPT // ZR):
        pltpu.sync_copy(zbuf, acc_sh.at[pl.ds(row0 + z * ZR, ZR)])
    plsc.subcore_barrier()

    ebase = sid * EPT

    def _fetch(p_hbm, c, b):
        base = ebase + c * C2
        pltpu.sync_copy(src_hbm.at[pl.ds(base, C2)], srcb[b])
        pltpu.sync_copy(dst_hbm.at[pl.ds(base, C2)], dstb[b])
        pltpu.async_copy(p_hbm.at[srcb[b]], rowsb[b], gsem[b])

    def _drain_scatter(p_hbm, b):
        pltpu.make_async_copy(p_hbm.at[srcb[b]], rowsb[b], gsem[b]).wait()
        pltpu.sync_copy(rowsb[b], acc_sh.at[dstb[b]], add=True)

    def _run_edges(p_hbm):
        _fetch(p_hbm, 0, 0)

        def _pair(kk, carry):
            for b in range(2):
                c = 2 * kk + b
                _fetch(p_hbm, c + 1, 1 - b)
                _drain_scatter(p_hbm, b)
            return carry
        lax.fori_loop(0, NCHUNK2 // 2 - 1, _pair, 0)
        _fetch(p_hbm, NCHUNK2 - 1, 1)
        _drain_scatter(p_hbm, 0)
        _drain_scatter(p_hbm, 1)

    @pl.when(cid == 0)
    def _core0():
        _run_edges(plo_hbm)

    @pl.when(cid == 1)
    def _core1():
        _run_edges(phi_hbm)

    plsc.subcore_barrier()

    @pl.when(cid == 0)
    def _dump0():
        pltpu.sync_copy(acc_sh.at[pl.ds(row0, NPT)],
                        acclo_hbm.at[pl.ds(row0, NPT)])

    @pl.when(cid == 1)
    def _dump1():
        pltpu.sync_copy(acc_sh.at[pl.ds(row0, NPT)],
                        acchi_hbm.at[pl.ds(row0, NPT)])


def _run_b2(src, dst, p_lo, p_hi):
    f = pl.kernel(
        _b2_body,
        out_type=[
            jax.ShapeDtypeStruct((NPAD, FH), jnp.float32),
            jax.ShapeDtypeStruct((NPAD, FH), jnp.float32),
        ],
        mesh=_sc_mesh(),
        compiler_params=_SC_PARAMS,
        scratch_types=[
            pltpu.VMEM((C2,), jnp.int32),
            pltpu.VMEM((C2,), jnp.int32),
            pltpu.VMEM((C2,), jnp.int32),
            pltpu.VMEM((C2,), jnp.int32),
            pltpu.VMEM((C2, FH), jnp.float32),
            pltpu.VMEM((C2, FH), jnp.float32),
            pltpu.VMEM((ZR, FH), jnp.float32),
            pltpu.VMEM_SHARED((NPAD, FH), jnp.float32),
            pltpu.SemaphoreType.DMA,
            pltpu.SemaphoreType.DMA,
        ],
    )
    return f(src, dst, p_lo, p_hi)


def _epilogue_body(a0_ref, a1_ref, a2_ref, a3_ref,
                   d0_ref, d1_ref, d2_ref, d3_ref,
                   eal_ref, eah_ref, qb_ref, selT_ref, out_ref, es_ref):
    den4 = jnp.concatenate([d0_ref[...][:, 0:1], d1_ref[...][:, 0:1],
                            d2_ref[...][:, 0:1], d3_ref[...][:, 0:1]], axis=1)
    den128 = jnp.dot(den4, selT_ref[...], preferred_element_type=jnp.float32)
    deg128 = jnp.broadcast_to(d0_ref[...][:, 1:2], (BN, F))
    den_safe = jnp.where(den128 > 0, den128, 1.0)
    oa = jnp.concatenate([a0_ref[...], a1_ref[...], a2_ref[...], a3_ref[...]],
                         axis=1)
    ea = jnp.concatenate([eal_ref[...], eah_ref[...]], axis=1)
    out_ref[...] = oa / den_safe
    es_ref[...] = (ea + deg128 * qb_ref[...]) / jnp.maximum(deg128, 1.0)


def _run_epilogue(accs, dds, eal, eah, qb, selT):
    return pl.pallas_call(
        _epilogue_body,
        grid=(N // BN,),
        in_specs=[
            pl.BlockSpec((BN, HD), lambda i: (i, 0)),
            pl.BlockSpec((BN, HD), lambda i: (i, 0)),
            pl.BlockSpec((BN, HD), lambda i: (i, 0)),
            pl.BlockSpec((BN, HD), lambda i: (i, 0)),
            pl.BlockSpec((BN, 8), lambda i: (i, 0)),
            pl.BlockSpec((BN, 8), lambda i: (i, 0)),
            pl.BlockSpec((BN, 8), lambda i: (i, 0)),
            pl.BlockSpec((BN, 8), lambda i: (i, 0)),
            pl.BlockSpec((BN, FH), lambda i: (i, 0)),
            pl.BlockSpec((BN, FH), lambda i: (i, 0)),
            pl.BlockSpec((BN, F), lambda i: (i, 0)),
            pl.BlockSpec((4, F), lambda i: (0, 0)),
        ],
        out_specs=[
            pl.BlockSpec((BN, F), lambda i: (i, 0)),
            pl.BlockSpec((BN, F), lambda i: (i, 0)),
        ],
        out_shape=[
            jax.ShapeDtypeStruct((N, F), jnp.float32),
            jax.ShapeDtypeStruct((N, F), jnp.float32),
        ],
    )(*accs, *dds, eal, eah, qb, selT)


@jax.jit
def kernel(x, edge_index, W_r, b_r, W_l, b_l, W_a, b_a):
    src = edge_index[0]
    dst = edge_index[1]
    Wr1 = W_r[:D]
    Wr2 = W_r[D:]
    w1 = jnp.tile(W_a[0:HD, 0], HEAD)
    w2 = jnp.tile(W_a[HD:2 * HD, 0], HEAD)
    w3 = jnp.tile(W_a[2 * HD:, 0], HEAD)
    ba_row = jnp.zeros((F,), jnp.float32).at[0].set(b_a[0])
    wa_pack = jnp.stack([w1, w2, w3, ba_row], axis=0)
    didx = jnp.arange(F) // HD
    sel = jnp.stack([(didx == (hh % HEAD)).astype(jnp.float32)
                     for hh in range(8)], axis=1)
    selT = jnp.stack([(didx == hh).astype(jnp.float32)
                      for hh in range(HEAD)], axis=0)

    h, p, qb, st = _run_prologue(x, W_l, b_l, Wr1, Wr2, b_r, wa_pack, sel)
    h0, h1, h2, h3 = (h[:, i * HD:(i + 1) * HD] for i in range(HEAD))
    p_lo = p[:, :FH]
    p_hi = p[:, FH:]
    a0, a1, dd0, dd1 = _run_b1(0, src, dst, h0, h1, st)
    a2, a3, dd2, dd3 = _run_b1(1, src, dst, h2, h3, st)
    eal, eah = _run_b2(src, dst, p_lo, p_hi)
    out, es = _run_epilogue(
        [a0[:N], a1[:N], a2[:N], a3[:N]],
        [dd0[:N], dd1[:N], dd2[:N], dd3[:N]],
        eal[:N], eah[:N], qb, selT)
    return (es, out, h)


# speedup vs baseline: 110.0771x; 1.0020x over previous
import jax
import jax.numpy as jnp
from jax import lax
from jax.experimental import pallas as pl
from jax.experimental.pallas import tpu as pltpu
from jax.experimental.pallas import tpu_sc as plsc

N = 10000
E = 320000
D = 128
HEAD = 4
HD = 32
F = HEAD * HD
FH = F // 2

NC = 2
NS = 16

EPT = E // NS
C = 160
NCHUNK = EPT // C
C2 = 400
NCHUNK2 = EPT // C2
NPAD = 10240
NPT = NPAD // NS
ZR = 128

BN = 1000

_SC_PARAMS = pltpu.CompilerParams(needs_layout_passes=False,
                                  use_tc_tiling_on_sc=False)


def _sc_mesh():
    return plsc.VectorSubcoreMesh(core_axis_name="c", subcore_axis_name="s",
                                  num_cores=NC, num_subcores=NS)


def _prologue_body(x_ref, wl_ref, bl_ref, wr1_ref, wr2_ref, br_ref, wa_ref,
                   sel_ref, h_ref, p_ref, qb_ref, st_ref):
    xb = x_ref[...]
    h = jnp.dot(xb, wl_ref[...], preferred_element_type=jnp.float32) + bl_ref[...]
    p = jnp.dot(xb, wr1_ref[...], preferred_element_type=jnp.float32)
    qb = jnp.dot(xb, wr2_ref[...], preferred_element_type=jnp.float32) + br_ref[...]
    h_ref[...] = h
    p_ref[...] = p
    qb_ref[...] = qb
    wa = wa_ref[...]
    sel = sel_ref[...]
    s = jnp.dot(h * wa[0] + p * wa[2], sel,
                preferred_element_type=jnp.float32)
    t = jnp.dot(h * wa[1] + qb * wa[2], sel,
                preferred_element_type=jnp.float32)
    st = jnp.concatenate([s[:, :4] + wa[3, 0], t[:, 4:8]], axis=1)
    st_ref[...] = st


def _run_prologue(x, W_l, b_l, Wr1, Wr2, b_r, wa_pack, sel):
    return pl.pallas_call(
        _prologue_body,
        grid=(N // BN,),
        in_specs=[
            pl.BlockSpec((BN, D), lambda i: (i, 0)),
            pl.BlockSpec((D, F), lambda i: (0, 0)),
            pl.BlockSpec((F,), lambda i: (0,)),
            pl.BlockSpec((D, F), lambda i: (0, 0)),
            pl.BlockSpec((D, F), lambda i: (0, 0)),
            pl.BlockSpec((F,), lambda i: (0,)),
            pl.BlockSpec((4, F), lambda i: (0, 0)),
            pl.BlockSpec((F, 8), lambda i: (0, 0)),
        ],
        out_specs=[
            pl.BlockSpec((BN, F), lambda i: (i, 0)),
            pl.BlockSpec((BN, F), lambda i: (i, 0)),
            pl.BlockSpec((BN, F), lambda i: (i, 0)),
            pl.BlockSpec((BN, 8), lambda i: (i, 0)),
        ],
        out_shape=[
            jax.ShapeDtypeStruct((N, F), jnp.float32),
            jax.ShapeDtypeStruct((N, F), jnp.float32),
            jax.ShapeDtypeStruct((N, F), jnp.float32),
            jax.ShapeDtypeStruct((N, 8), jnp.float32),
        ],
    )(x, W_l, b_l, Wr1, Wr2, b_r, wa_pack, sel)


def _zero_narrow(ref, nelem):
    iota16 = lax.iota(jnp.int32, 16)
    zval = jnp.zeros((16,), jnp.float32)

    def _z(i, carry):
        idx = iota16 + i * 16
        plsc.store_scatter(ref, [lax.shift_right_logical(idx, 3),
                                 lax.bitwise_and(idx, 7)], zval)
        return carry
    lax.fori_loop(0, nelem // 16, _z, 0)


def _make_b1_body(khalf):
    def _b1_body(src_hbm, dst_hbm, ha_hbm, hb_hbm, st_hbm,
                 acca_hbm, accb_hbm, dda_hbm, ddb_hbm,
                 tbl, src0, src1, dst0, dst1, rows0, rows1, den0, den1,
                 zbuf, zbuf2, acc_sh, dd_sh, gsem0, gsem1, ssem0, ssem1):
        cid = lax.axis_index("c")
        sid = lax.axis_index("s")
        row0 = sid * NPT
        srcb = (src0, src1)
        dstb = (dst0, dst1)
        rowsb = (rows0, rows1)
        denb = (den0, den1)
        gsem = (gsem0, gsem1)
        ssemb = (ssem0, ssem1)

        def _zrow(r, carry):
            for cb in range(HD // 16):
                zbuf[r, pl.ds(cb * 16, 16)] = jnp.zeros((16,), jnp.float32)
            return carry
        lax.fori_loop(0, ZR, _zrow, 0)
        _zero_narrow(zbuf2, ZR * 8)
        for z in range(NPT // ZR):
            pltpu.sync_copy(zbuf, acc_sh.at[pl.ds(row0 + z * ZR, ZR)])
            pltpu.sync_copy(zbuf2, dd_sh.at[pl.ds(row0 + z * ZR, ZR)])
        pltpu.sync_copy(st_hbm, tbl)
        plsc.subcore_barrier()

        ebase = sid * EPT
        iota16 = lax.iota(jnp.int32, 16)
        if khalf == 0:
            degval = jnp.broadcast_to(jnp.where(cid == 0, 1.0, 0.0), (16,))
        else:
            degval = jnp.zeros((16,), jnp.float32)
        col1 = jnp.full((16,), 1, jnp.int32)
        for b in range(2):
            _zero_narrow(denb[b], C * 8)

            def _dinit(r, carry, _d=denb[b]):
                plsc.store_scatter(_d, [iota16 + r * 16, col1], degval)
                return carry
            lax.fori_loop(0, C // 16, _dinit, 0)

        col0 = jnp.full((16,), 0, jnp.int32)
        head = 2 * khalf + cid
        hs = jnp.full((16,), 0, jnp.int32) + head
        ht = jnp.full((16,), HEAD, jnp.int32) + head

        def _compute(b):
            srcq, dstq, rows, denbuf = srcb[b], dstb[b], rowsb[b], denb[b]
            for g in range(C // 16):
                r0 = g * 16
                s16 = srcq[pl.ds(r0, 16)]
                d16 = dstq[pl.ds(r0, 16)]
                sv = plsc.load_gather(tbl, [s16, hs])
                tv = plsc.load_gather(tbl, [d16, ht])
                a = sv + tv
                a = jnp.where(a > 0, a, a * 0.01)
                ev = jnp.exp(a)
                plsc.store_scatter(denbuf, [iota16 + r0, col0], ev)
                for j in range(16):
                    row = r0 + j
                    w = jnp.broadcast_to(ev[j], (16,))
                    rows[row, pl.ds(0, 16)] = rows[row, pl.ds(0, 16)] * w
                    rows[row, pl.ds(16, 16)] = rows[row, pl.ds(16, 16)] * w
                pltpu.async_copy(rows.at[pl.ds(r0, 16)], acc_sh.at[d16],
                                 ssemb[b], add=True)
                pltpu.async_copy(denbuf.at[pl.ds(r0, 16)], dd_sh.at[d16],
                                 ssemb[b], add=True)

        def _wait_scatter(b):
            pltpu.make_async_copy(rowsb[b], acc_sh.at[dstb[b]],
                                  ssemb[b]).wait()
            pltpu.make_async_copy(denb[b], dd_sh.at[dstb[b]], ssemb[b]).wait()

        def _fetch(h_hbm, c, b):
            base = ebase + c * C
            pltpu.sync_copy(src_hbm.at[pl.ds(base, C)], srcb[b])
            pltpu.sync_copy(dst_hbm.at[pl.ds(base, C)], dstb[b])
            pltpu.async_copy(h_hbm.at[srcb[b]], rowsb[b], gsem[b])

        def _run_edges(h_hbm):
            _fetch(h_hbm, 0, 0)

            def _pair(kk, carry):
                for b in range(2):
                    c = 2 * kk + b
                    if b == 0:
                        @pl.when(kk > 0)
                        def _():
                            _wait_scatter(1)
                    else:
                        _wait_scatter(0)
                    _fetch(h_hbm, c + 1, 1 - b)
                    pltpu.make_async_copy(h_hbm.at[srcb[b]], rowsb[b],
                                          gsem[b]).wait()
                    _compute(b)
                return carry
            lax.fori_loop(0, NCHUNK // 2, _pair, 0)
            pltpu.make_async_copy(h_hbm.at[srcb[0]], rowsb[0], gsem[0]).wait()
            _compute(0)
            _wait_scatter(1)
            _wait_scatter(0)

        @pl.when(cid == 0)
        def _core0():
            _run_edges(ha_hbm)

        @pl.when(cid == 1)
        def _core1():
            _run_edges(hb_hbm)

        plsc.subcore_barrier()

        @pl.when(cid == 0)
        def _dump0():
            pltpu.sync_copy(acc_sh.at[pl.ds(row0, NPT)],
                            acca_hbm.at[pl.ds(row0, NPT)])
            pltpu.sync_copy(dd_sh.at[pl.ds(row0, NPT)],
                            dda_hbm.at[pl.ds(row0, NPT)])

        @pl.when(cid == 1)
        def _dump1():
            pltpu.sync_copy(acc_sh.at[pl.ds(row0, NPT)],
                            accb_hbm.at[pl.ds(row0, NPT)])
            pltpu.sync_copy(dd_sh.at[pl.ds(row0, NPT)],
                            ddb_hbm.at[pl.ds(row0, NPT)])

    return _b1_body


def _run_b1(khalf, src, dst, h_a, h_b, st):
    f = pl.kernel(
        _make_b1_body(khalf),
        out_type=[
            jax.ShapeDtypeStruct((NPAD, HD), jnp.float32),
            jax.ShapeDtypeStruct((NPAD, HD), jnp.float32),
            jax.ShapeDtypeStruct((NPAD, 8), jnp.float32),
            jax.ShapeDtypeStruct((NPAD, 8), jnp.float32),
        ],
        mesh=_sc_mesh(),
        compiler_params=_SC_PARAMS,
        scratch_types=[
            pltpu.VMEM((N, 8), jnp.float32),
            pltpu.VMEM((C,), jnp.int32),
            pltpu.VMEM((C,), jnp.int32),
            pltpu.VMEM((C,), jnp.int32),
            pltpu.VMEM((C,), jnp.int32),
            pltpu.VMEM((C, HD), jnp.float32),
            pltpu.VMEM((C, HD), jnp.float32),
            pltpu.VMEM((C, 8), jnp.float32),
            pltpu.VMEM((C, 8), jnp.float32),
            pltpu.VMEM((ZR, HD), jnp.float32),
            pltpu.VMEM((ZR, 8), jnp.float32),
            pltpu.VMEM_SHARED((NPAD, HD), jnp.float32),
            pltpu.VMEM_SHARED((NPAD, 8), jnp.float32),
            pltpu.SemaphoreType.DMA,
            pltpu.SemaphoreType.DMA,
            pltpu.SemaphoreType.DMA,
            pltpu.SemaphoreType.DMA,
        ],
    )
    return f(src, dst, h_a, h_b, st)


def _b2_body(src_hbm, dst_hbm, plo_hbm, phi_hbm,
             acclo_hbm, acchi_hbm,
             src0, src1, dst0, dst1, rows0, rows1, zbuf, acc_sh,
             gsem0, gsem1):
    cid = lax.axis_index("c")
    sid = lax.axis_index("s")
    row0 = sid * NPT
    srcb = (src0, src1)
    dstb = (dst0, dst1)
    rowsb = (rows0, rows1)
    gsem = (gsem0, gsem1)

    def _zrow(r, carry):
        for cb in range(FH // 16):
            zbuf[r, pl.ds(cb * 16, 16)] = jnp.zeros((16,), jnp.float32)
        return carry
    lax.fori_loop(0, ZR, _zrow, 0)
    for z in range(NPT // ZR):
        pltpu.sync_copy(zbuf, acc_sh.at[pl.ds(row0 + z * ZR, ZR)])
    plsc.subcore_barrier()

    ebase = sid * EPT

    def _fetch(p_hbm, c, b):
        base = ebase + c * C2
        pltpu.sync_copy(src_hbm.at[pl.ds(base, C2)], srcb[b])
        pltpu.sync_copy(dst_hbm.at[pl.ds(base, C2)], dstb[b])
        pltpu.async_copy(p_hbm.at[srcb[b]], rowsb[b], gsem[b])

    def _drain_scatter(p_hbm, b):
        pltpu.make_async_copy(p_hbm.at[srcb[b]], rowsb[b], gsem[b]).wait()
        pltpu.sync_copy(rowsb[b], acc_sh.at[dstb[b]], add=True)

    def _run_edges(p_hbm):
        _fetch(p_hbm, 0, 0)

        def _pair(kk, carry):
            for b in range(2):
                c = 2 * kk + b
                _fetch(p_hbm, c + 1, 1 - b)
                _drain_scatter(p_hbm, b)
            return carry
        lax.fori_loop(0, NCHUNK2 // 2 - 1, _pair, 0)
        _fetch(p_hbm, NCHUNK2 - 1, 1)
        _drain_scatter(p_hbm, 0)
        _drain_scatter(p_hbm, 1)

    @pl.when(cid == 0)
    def _core0():
        _run_edges(plo_hbm)

    @pl.when(cid == 1)
    def _core1():
        _run_edges(phi_hbm)

    plsc.subcore_barrier()

    @pl.when(cid == 0)
    def _dump0():
        pltpu.sync_copy(acc_sh.at[pl.ds(row0, NPT)],
                        acclo_hbm.at[pl.ds(row0, NPT)])

    @pl.when(cid == 1)
    def _dump1():
        pltpu.sync_copy(acc_sh.at[pl.ds(row0, NPT)],
                        acchi_hbm.at[pl.ds(row0, NPT)])


def _run_b2(src, dst, p_lo, p_hi):
    f = pl.kernel(
        _b2_body,
        out_type=[
            jax.ShapeDtypeStruct((NPAD, FH), jnp.float32),
            jax.ShapeDtypeStruct((NPAD, FH), jnp.float32),
        ],
        mesh=_sc_mesh(),
        compiler_params=_SC_PARAMS,
        scratch_types=[
            pltpu.VMEM((C2,), jnp.int32),
            pltpu.VMEM((C2,), jnp.int32),
            pltpu.VMEM((C2,), jnp.int32),
            pltpu.VMEM((C2,), jnp.int32),
            pltpu.VMEM((C2, FH), jnp.float32),
            pltpu.VMEM((C2, FH), jnp.float32),
            pltpu.VMEM((ZR, FH), jnp.float32),
            pltpu.VMEM_SHARED((NPAD, FH), jnp.float32),
            pltpu.SemaphoreType.DMA,
            pltpu.SemaphoreType.DMA,
        ],
    )
    return f(src, dst, p_lo, p_hi)


def _epilogue_body(a0_ref, a1_ref, a2_ref, a3_ref,
                   d0_ref, d1_ref, d2_ref, d3_ref,
                   eal_ref, eah_ref, qb_ref, selT_ref, out_ref, es_ref):
    den4 = jnp.concatenate([d0_ref[...][:, 0:1], d1_ref[...][:, 0:1],
                            d2_ref[...][:, 0:1], d3_ref[...][:, 0:1]], axis=1)
    den128 = jnp.dot(den4, selT_ref[...], preferred_element_type=jnp.float32)
    deg128 = jnp.broadcast_to(d0_ref[...][:, 1:2], (BN, F))
    den_safe = jnp.where(den128 > 0, den128, 1.0)
    oa = jnp.concatenate([a0_ref[...], a1_ref[...], a2_ref[...], a3_ref[...]],
                         axis=1)
    ea = jnp.concatenate([eal_ref[...], eah_ref[...]], axis=1)
    out_ref[...] = oa / den_safe
    es_ref[...] = (ea + deg128 * qb_ref[...]) / jnp.maximum(deg128, 1.0)


def _run_epilogue(accs, dds, eal, eah, qb, selT):
    return pl.pallas_call(
        _epilogue_body,
        grid=(N // BN,),
        in_specs=[
            pl.BlockSpec((BN, HD), lambda i: (i, 0)),
            pl.BlockSpec((BN, HD), lambda i: (i, 0)),
            pl.BlockSpec((BN, HD), lambda i: (i, 0)),
            pl.BlockSpec((BN, HD), lambda i: (i, 0)),
            pl.BlockSpec((BN, 8), lambda i: (i, 0)),
            pl.BlockSpec((BN, 8), lambda i: (i, 0)),
            pl.BlockSpec((BN, 8), lambda i: (i, 0)),
            pl.BlockSpec((BN, 8), lambda i: (i, 0)),
            pl.BlockSpec((BN, FH), lambda i: (i, 0)),
            pl.BlockSpec((BN, FH), lambda i: (i, 0)),
            pl.BlockSpec((BN, F), lambda i: (i, 0)),
            pl.BlockSpec((4, F), lambda i: (0, 0)),
        ],
        out_specs=[
            pl.BlockSpec((BN, F), lambda i: (i, 0)),
            pl.BlockSpec((BN, F), lambda i: (i, 0)),
        ],
        out_shape=[
            jax.ShapeDtypeStruct((N, F), jnp.float32),
            jax.ShapeDtypeStruct((N, F), jnp.float32),
        ],
    )(*accs, *dds, eal, eah, qb, selT)


@jax.jit
def kernel(x, edge_index, W_r, b_r, W_l, b_l, W_a, b_a):
    src = edge_index[0]
    dst = edge_index[1]
    Wr1 = W_r[:D]
    Wr2 = W_r[D:]
    w1 = jnp.tile(W_a[0:HD, 0], HEAD)
    w2 = jnp.tile(W_a[HD:2 * HD, 0], HEAD)
    w3 = jnp.tile(W_a[2 * HD:, 0], HEAD)
    ba_row = jnp.zeros((F,), jnp.float32).at[0].set(b_a[0])
    wa_pack = jnp.stack([w1, w2, w3, ba_row], axis=0)
    didx = jnp.arange(F) // HD
    sel = jnp.stack([(didx == (hh % HEAD)).astype(jnp.float32)
                     for hh in range(8)], axis=1)
    selT = jnp.stack([(didx == hh).astype(jnp.float32)
                      for hh in range(HEAD)], axis=0)

    h, p, qb, st = _run_prologue(x, W_l, b_l, Wr1, Wr2, b_r, wa_pack, sel)
    h0, h1, h2, h3 = (h[:, i * HD:(i + 1) * HD] for i in range(HEAD))
    p_lo = p[:, :FH]
    p_hi = p[:, FH:]
    a0, a1, dd0, dd1 = _run_b1(0, src, dst, h0, h1, st)
    a2, a3, dd2, dd3 = _run_b1(1, src, dst, h2, h3, st)
    eal, eah = _run_b2(src, dst, p_lo, p_hi)
    out, es = _run_epilogue(
        [a0[:N], a1[:N], a2[:N], a3[:N]],
        [dd0[:N], dd1[:N], dd2[:N], dd3[:N]],
        eal[:N], eah[:N], qb, selT)
    return (es, out, h)
